# baseline (speedup 1.0000x reference)
import functools

import jax
import jax.numpy as jnp
from jax import lax
from jax.experimental import pallas as pl
from jax.experimental.pallas import tpu as pltpu
from jax.experimental.pallas import tpu_sc as plsc

MAXD = 200
N = 100000
D = 64
NC, NS, L = 2, 16, 16
NW = NC * NS
BPW = 3200
BPAD = BPW * NW
C = 640
NCH = BPW // C
KB = C // 128

_mesh = plsc.VectorSubcoreMesh(core_axis_name="c", subcore_axis_name="s")


@functools.partial(
    pl.kernel,
    out_type=jax.ShapeDtypeStruct((BPAD, D), jnp.float32),
    mesh=_mesh,
    scratch_types=[
        pltpu.VMEM((C,), jnp.int32),
        pltpu.VMEM((C,), jnp.int32),
        pltpu.VMEM((C, D), jnp.float32),
        pltpu.VMEM((C, D), jnp.float32),
        pltpu.SemaphoreType.DMA,
        pltpu.SemaphoreType.DMA,
    ],
    compiler_params=pltpu.CompilerParams(use_tc_tiling_on_sc=False),
)
def _encode(tt_hbm, dt_hbm, nt_hbm, dp_hbm, out_hbm,
            nt_v, d_v, rows_t, rows_d, sem_t, sem_d):
    wid = lax.axis_index("s") * NC + lax.axis_index("c")
    base_w = wid * BPW

    def chunk(ch, carry):
        base = pl.multiple_of(base_w + ch * C, C)
        pltpu.sync_copy(nt_hbm.at[pl.ds(base, C)], nt_v)
        pltpu.sync_copy(dp_hbm.at[pl.ds(base, C)], d_v)

        for i in range(C // L):
            sl = pl.ds(i * L, L)
            d_v[sl] = jnp.minimum(d_v[sl], MAXD)

        cps = []
        for k in range(KB):
            cps.append(pltpu.async_copy(
                tt_hbm.at[nt_v.at[pl.ds(k * 128, 128)]],
                rows_t.at[pl.ds(k * 128, 128)], sem_t))
            cps.append(pltpu.async_copy(
                dt_hbm.at[d_v.at[pl.ds(k * 128, 128)]],
                rows_d.at[pl.ds(k * 128, 128)], sem_d))
        for cp in cps:
            cp.wait()

        def add(r, c2):
            for c4 in range(D // L):
                sl = (r, pl.ds(c4 * L, L))
                rows_t[sl] = rows_t[sl] + rows_d[sl]
            return c2
        lax.fori_loop(0, C, add, 0)

        pltpu.sync_copy(rows_t, out_hbm.at[pl.ds(base, C)])
        return carry

    lax.fori_loop(0, NCH, chunk, 0)


@jax.jit
def kernel(node_type, depth, type_table, depth_table):
    nt = jnp.zeros((BPAD,), jnp.int32).at[:N].set(node_type.astype(jnp.int32))
    dp = jnp.zeros((BPAD,), jnp.int32).at[:N].set(depth.astype(jnp.int32))
    out = _encode(type_table, depth_table, nt, dp)
    return out[:N]


# speedup vs baseline: 1.0036x; 1.0036x over previous
import functools

import jax
import jax.numpy as jnp
from jax import lax
from jax.experimental import pallas as pl
from jax.experimental.pallas import tpu as pltpu
from jax.experimental.pallas import tpu_sc as plsc

MAXD = 200
N = 100000
D = 64
NC, NS, L = 2, 16, 16
NW = NC * NS
BPW = 3200
BPAD = BPW * NW
C = 640
NCH = BPW // C
KB = C // 128

_mesh = plsc.VectorSubcoreMesh(core_axis_name="c", subcore_axis_name="s")


@functools.partial(
    pl.kernel,
    out_type=jax.ShapeDtypeStruct((BPAD, D), jnp.float32),
    mesh=_mesh,
    scratch_types=[
        pltpu.VMEM((C,), jnp.int32),
        pltpu.VMEM((C,), jnp.int32),
        pltpu.VMEM((C, D), jnp.float32),
        pltpu.SemaphoreType.DMA,
        pltpu.SemaphoreType.DMA,
    ],
    compiler_params=pltpu.CompilerParams(use_tc_tiling_on_sc=False),
)
def _encode(tt_hbm, dt_hbm, nt_hbm, dp_hbm, out_hbm,
            nt_v, d_v, rows_t, sem_t, sem_d):
    wid = lax.axis_index("s") * NC + lax.axis_index("c")
    base_w = wid * BPW

    def chunk(ch, carry):
        base = pl.multiple_of(base_w + ch * C, C)
        pltpu.sync_copy(nt_hbm.at[pl.ds(base, C)], nt_v)
        pltpu.sync_copy(dp_hbm.at[pl.ds(base, C)], d_v)

        for i in range(C // L):
            sl = pl.ds(i * L, L)
            d_v[sl] = jnp.minimum(d_v[sl], MAXD)

        cps = []
        for k in range(KB):
            cps.append(pltpu.async_copy(
                tt_hbm.at[nt_v.at[pl.ds(k * 128, 128)]],
                rows_t.at[pl.ds(k * 128, 128)], sem_t))
        dps = []
        for k in range(KB):
            cps[k].wait()
            dps.append(pltpu.async_copy(
                dt_hbm.at[d_v.at[pl.ds(k * 128, 128)]],
                rows_t.at[pl.ds(k * 128, 128)], sem_d, add=True))
        for dp in dps:
            dp.wait()

        pltpu.sync_copy(rows_t, out_hbm.at[pl.ds(base, C)])
        return carry

    lax.fori_loop(0, NCH, chunk, 0)


@jax.jit
def kernel(node_type, depth, type_table, depth_table):
    nt = jnp.zeros((BPAD,), jnp.int32).at[:N].set(node_type.astype(jnp.int32))
    dp = jnp.zeros((BPAD,), jnp.int32).at[:N].set(depth.astype(jnp.int32))
    out = _encode(type_table, depth_table, nt, dp)
    return out[:N]


# speedup vs baseline: 1.0120x; 1.0084x over previous
import functools

import jax
import jax.numpy as jnp
from jax import lax
from jax.experimental import pallas as pl
from jax.experimental.pallas import tpu as pltpu
from jax.experimental.pallas import tpu_sc as plsc

MAXD = 200
N = 100000
D = 64
NC, NS, L = 2, 16, 16
NW = NC * NS
BPW = 3200
BPAD = BPW * NW
C = 1600
NCH = BPW // C

_mesh = plsc.VectorSubcoreMesh(core_axis_name="c", subcore_axis_name="s")


@functools.partial(
    pl.kernel,
    out_type=jax.ShapeDtypeStruct((BPAD, D), jnp.float32),
    mesh=_mesh,
    scratch_types=[
        pltpu.VMEM((C,), jnp.int32),
        pltpu.VMEM((C,), jnp.int32),
        pltpu.VMEM((C, D), jnp.float32),
        pltpu.SemaphoreType.DMA,
        pltpu.SemaphoreType.DMA,
    ],
    compiler_params=pltpu.CompilerParams(use_tc_tiling_on_sc=False),
)
def _encode(tt_hbm, dt_hbm, nt_hbm, dp_hbm, out_hbm,
            nt_v, d_v, rows_t, sem_t, sem_d):
    wid = lax.axis_index("s") * NC + lax.axis_index("c")
    base_w = wid * BPW

    def chunk(ch, carry):
        base = pl.multiple_of(base_w + ch * C, C)
        pltpu.sync_copy(nt_hbm.at[pl.ds(base, C)], nt_v)
        pltpu.sync_copy(dp_hbm.at[pl.ds(base, C)], d_v)

        for i in range(C // L):
            sl = pl.ds(i * L, L)
            d_v[sl] = jnp.minimum(d_v[sl], MAXD)

        pltpu.async_copy(tt_hbm.at[nt_v], rows_t, sem_t).wait()
        pltpu.async_copy(dt_hbm.at[d_v], rows_t, sem_d, add=True).wait()

        pltpu.sync_copy(rows_t, out_hbm.at[pl.ds(base, C)])
        return carry

    lax.fori_loop(0, NCH, chunk, 0)


@jax.jit
def kernel(node_type, depth, type_table, depth_table):
    nt = jnp.zeros((BPAD,), jnp.int32).at[:N].set(node_type.astype(jnp.int32))
    dp = jnp.zeros((BPAD,), jnp.int32).at[:N].set(depth.astype(jnp.int32))
    out = _encode(type_table, depth_table, nt, dp)
    return out[:N]


# speedup vs baseline: 4.0325x; 3.9846x over previous
import functools

import jax
import jax.numpy as jnp
from jax import lax
from jax.experimental import pallas as pl
from jax.experimental.pallas import tpu as pltpu
from jax.experimental.pallas import tpu_sc as plsc

MAXD = 200
N = 100000
D = 64
NC, NS, L = 2, 16, 16
NW = NC * NS
BPW = 3200
BPAD = BPW * NW
C = 1600
NCH = BPW // C

_mesh = plsc.VectorSubcoreMesh(core_axis_name="c", subcore_axis_name="s")


@functools.partial(
    pl.kernel,
    out_type=jax.ShapeDtypeStruct((BPAD, D), jnp.float32),
    mesh=_mesh,
    scratch_types=[
        pltpu.VMEM((C,), jnp.int32),
        pltpu.VMEM((C,), jnp.int32),
        pltpu.VMEM((C, D), jnp.float32),
        pltpu.VMEM_SHARED((MAXD + 1, D), jnp.float32),
        pltpu.SemaphoreType.DMA,
        pltpu.SemaphoreType.DMA,
    ],
    compiler_params=pltpu.CompilerParams(use_tc_tiling_on_sc=False),
)
def _encode(tt_hbm, dt_hbm, nt_hbm, dp_hbm, out_hbm,
            nt_v, d_v, rows_t, dt_sp, sem_t, sem_d):
    sid = lax.axis_index("s")
    wid = sid * NC + lax.axis_index("c")
    base_w = wid * BPW

    @pl.when(sid == 0)
    def _():
        pltpu.sync_copy(dt_hbm, dt_sp)
    plsc.subcore_barrier()

    def chunk(ch, carry):
        base = pl.multiple_of(base_w + ch * C, C)
        pltpu.sync_copy(nt_hbm.at[pl.ds(base, C)], nt_v)
        pltpu.sync_copy(dp_hbm.at[pl.ds(base, C)], d_v)

        for i in range(C // L):
            sl = pl.ds(i * L, L)
            d_v[sl] = jnp.minimum(d_v[sl], MAXD)

        pltpu.async_copy(tt_hbm.at[nt_v], rows_t, sem_t).wait()
        pltpu.async_copy(dt_sp.at[d_v], rows_t, sem_d, add=True).wait()

        pltpu.sync_copy(rows_t, out_hbm.at[pl.ds(base, C)])
        return carry

    lax.fori_loop(0, NCH, chunk, 0)


@jax.jit
def kernel(node_type, depth, type_table, depth_table):
    pad = jnp.arange(BPAD - N, dtype=jnp.int32)
    nt = jnp.concatenate([node_type.astype(jnp.int32), pad])
    dp = jnp.concatenate([depth.astype(jnp.int32), pad])
    out = _encode(type_table, depth_table, nt, dp)
    return out[:N]


# speedup vs baseline: 4.7892x; 1.1876x over previous
import functools

import jax
import jax.numpy as jnp
from jax import lax
from jax.experimental import pallas as pl
from jax.experimental.pallas import tpu as pltpu
from jax.experimental.pallas import tpu_sc as plsc

MAXD = 200
N = 100000
D = 64
NC, NS, L = 2, 16, 16
NW = NC * NS
BPW = 3200
C = 1600
NCH = BPW // C
CT = N - (NW - 1) * BPW

_mesh = plsc.VectorSubcoreMesh(core_axis_name="c", subcore_axis_name="s")


@functools.partial(
    pl.kernel,
    out_type=jax.ShapeDtypeStruct((N, D), jnp.float32),
    mesh=_mesh,
    scratch_types=[
        pltpu.VMEM((C,), jnp.int32),
        pltpu.VMEM((C,), jnp.int32),
        pltpu.VMEM((C, D), jnp.float32),
        pltpu.VMEM_SHARED((MAXD + 1, D), jnp.float32),
        pltpu.SemaphoreType.DMA,
        pltpu.SemaphoreType.DMA,
    ],
    compiler_params=pltpu.CompilerParams(use_tc_tiling_on_sc=False),
)
def _encode(tt_hbm, dt_hbm, nt_hbm, dp_hbm, out_hbm,
            nt_v, d_v, rows_t, dt_sp, sem_t, sem_d):
    sid = lax.axis_index("s")
    wid = sid * NC + lax.axis_index("c")
    base_w = wid * BPW

    @pl.when(sid == 0)
    def _():
        pltpu.sync_copy(dt_hbm, dt_sp)
    plsc.subcore_barrier()

    def do_chunk(base, c, nt_vc, d_vc, rows_c):
        base = pl.multiple_of(base, 8)
        pltpu.sync_copy(nt_hbm.at[pl.ds(base, c)], nt_vc)
        pltpu.sync_copy(dp_hbm.at[pl.ds(base, c)], d_vc)
        for i in range(c // L):
            sl = pl.ds(i * L, L)
            d_vc[sl] = jnp.minimum(d_vc[sl], MAXD)
        pltpu.async_copy(tt_hbm.at[nt_vc], rows_c, sem_t).wait()
        pltpu.async_copy(dt_sp.at[d_vc], rows_c, sem_d, add=True).wait()
        pltpu.sync_copy(rows_c, out_hbm.at[pl.ds(base, c)])

    for ch in range(NCH):
        base = base_w + ch * C

        @pl.when(base + C <= N)
        def _():
            do_chunk(base, C, nt_v, d_v, rows_t)

    @pl.when(wid == NW - 1)
    def _():
        do_chunk((NW - 1) * BPW, CT,
                 nt_v.at[pl.ds(0, CT)], d_v.at[pl.ds(0, CT)],
                 rows_t.at[pl.ds(0, CT)])


@jax.jit
def kernel(node_type, depth, type_table, depth_table):
    return _encode(type_table, depth_table,
                   node_type.astype(jnp.int32), depth.astype(jnp.int32))


# speedup vs baseline: 4.7962x; 1.0014x over previous
import functools

import jax
import jax.numpy as jnp
from jax import lax
from jax.experimental import pallas as pl
from jax.experimental.pallas import tpu as pltpu
from jax.experimental.pallas import tpu_sc as plsc

MAXD = 200
N = 100000
D = 64
NC, NS, L = 2, 16, 16
NW = NC * NS
BPW = 3200
C = 1600
NCH = BPW // C
CT = N - (NW - 1) * BPW

_mesh = plsc.VectorSubcoreMesh(core_axis_name="c", subcore_axis_name="s")


@functools.partial(
    pl.kernel,
    out_type=jax.ShapeDtypeStruct((N, D), jnp.float32),
    mesh=_mesh,
    scratch_types=[
        pltpu.VMEM((C,), jnp.int32),
        pltpu.VMEM((C,), jnp.int32),
        pltpu.VMEM((C, D), jnp.float32),
        pltpu.VMEM_SHARED((MAXD + 1, D), jnp.float32),
        pltpu.SemaphoreType.DMA,
        pltpu.SemaphoreType.DMA,
    ],
    compiler_params=pltpu.CompilerParams(use_tc_tiling_on_sc=False,
                                         needs_layout_passes=False),
)
def _encode(tt_hbm, dt_hbm, nt_hbm, dp_hbm, out_hbm,
            nt_v, d_v, rows_t, dt_sp, sem_t, sem_d):
    sid = lax.axis_index("s")
    wid = sid * NC + lax.axis_index("c")
    base_w = wid * BPW

    @pl.when(sid == 0)
    def _():
        pltpu.sync_copy(dt_hbm, dt_sp)
    plsc.subcore_barrier()

    def do_chunk(base, c, nt_vc, d_vc, rows_c):
        base = pl.multiple_of(base, 8)
        pltpu.sync_copy(nt_hbm.at[pl.ds(base, c)], nt_vc)
        pltpu.sync_copy(dp_hbm.at[pl.ds(base, c)], d_vc)
        for i in range(c // L):
            sl = pl.ds(i * L, L)
            d_vc[sl] = jnp.minimum(d_vc[sl], MAXD)
        pltpu.async_copy(tt_hbm.at[nt_vc], rows_c, sem_t).wait()
        pltpu.async_copy(dt_sp.at[d_vc], rows_c, sem_d, add=True).wait()
        pltpu.sync_copy(rows_c, out_hbm.at[pl.ds(base, c)])

    for ch in range(NCH):
        base = base_w + ch * C

        @pl.when(base + C <= N)
        def _():
            do_chunk(base, C, nt_v, d_v, rows_t)

    @pl.when(wid == NW - 1)
    def _():
        do_chunk((NW - 1) * BPW, CT,
                 nt_v.at[pl.ds(0, CT)], d_v.at[pl.ds(0, CT)],
                 rows_t.at[pl.ds(0, CT)])


@jax.jit
def kernel(node_type, depth, type_table, depth_table):
    return _encode(type_table, depth_table,
                   node_type.astype(jnp.int32), depth.astype(jnp.int32))
